# baseline (speedup 1.0000x reference)
import functools

import jax
import jax.numpy as jnp
from jax import lax
from jax.experimental import pallas as pl
from jax.experimental.pallas import tpu as pltpu
from jax.experimental.pallas import tpu_sc as plsc

VOCAB = 1000000
DIM = 64
B = 16384
L = 50
NC = 2
NS = 16
NW = NC * NS
LANES = 16
BPW = B // NW
BCHUNK = 256
NCHUNK = L * BPW // BCHUNK
BSTRIPE = 384
NFULL = VOCAB // BSTRIPE
TAILC = VOCAB - NFULL * BSTRIPE

_COMPILER_PARAMS = pltpu.CompilerParams(
    use_tc_tiling_on_sc=True, needs_layout_passes=False,
    disable_bounds_checks=True)


def _make_phase_a():
    mesh = plsc.VectorSubcoreMesh(
        core_axis_name="c", subcore_axis_name="s",
        num_cores=NC, num_subcores=NS,
    )

    @functools.partial(
        pl.kernel,
        mesh=mesh,
        compiler_params=_COMPILER_PARAMS,
        out_type=jax.ShapeDtypeStruct((VOCAB // 2, 2 * DIM), jnp.float32),
        scratch_types=[
            pltpu.VMEM((DIM, BSTRIPE), jnp.float32),
            pltpu.VMEM((DIM, BSTRIPE), jnp.float32),
            pltpu.VMEM((BSTRIPE,), jnp.float32),
            pltpu.VMEM((BSTRIPE,), jnp.float32),
            pltpu.VMEM((BSTRIPE // 2, 2 * DIM), jnp.float32),
            pltpu.VMEM((BSTRIPE // 2, 2 * DIM), jnp.float32),
            pltpu.SemaphoreType.DMA,
            pltpu.SemaphoreType.DMA,
            pltpu.SemaphoreType.DMA,
            pltpu.SemaphoreType.DMA,
            pltpu.SemaphoreType.DMA,
            pltpu.SemaphoreType.DMA,
        ],
    )
    def ka(tt_hbm, maskf_hbm, tail_hbm, w2_hbm, in0, in1, mv0, mv1,
           ob0, ob1, sem_i0, sem_i1, sem_v0, sem_v1, sem_o0, sem_o1):
        INB = (in0, in1)
        MVB = (mv0, mv1)
        OUTB = (ob0, ob1)
        SEM_I = (sem_i0, sem_i1)
        SEM_V = (sem_v0, sem_v1)
        SEM_O = (sem_o0, sem_o1)
        wid = lax.axis_index("s") * NC + lax.axis_index("c")
        lane = lax.iota(jnp.int32, LANES)

        def fire(c, b):
            pltpu.async_copy(tt_hbm.at[:, pl.ds(c * BSTRIPE, BSTRIPE)],
                             INB[b], SEM_I[b])
            pltpu.async_copy(maskf_hbm.at[pl.ds(c * BSTRIPE, BSTRIPE)],
                             MVB[b], SEM_V[b])

        def recv(c, b):
            pltpu.make_async_copy(tt_hbm.at[:, pl.ds(c * BSTRIPE, BSTRIPE)],
                                  INB[b], SEM_I[b]).wait()
            pltpu.make_async_copy(maskf_hbm.at[pl.ds(c * BSTRIPE, BSTRIPE)],
                                  MVB[b], SEM_V[b]).wait()

        def transp(b, npb):
            inb, mvb, outb = INB[b], MVB[b], OUTB[b]

            def pb_body(pb, carry):
                prow = pb * LANES + lane
                cvec = 32 * pb + 2 * lane
                mval = (plsc.load_gather(mvb, [cvec]),
                        plsc.load_gather(mvb, [cvec + 1]))
                for kk in range(LANES):
                    diag = (lane + kk) & (LANES - 1)
                    vs = []
                    for hb in (0, 1):
                        for cq in range(4):
                            vs.append((hb, cq, plsc.load_gather(
                                inb, [cq * LANES + diag, cvec + hb])))
                    for hb, cq, v in vs:
                        plsc.store_scatter(
                            outb, [prow, (4 * hb + cq) * LANES + diag],
                            v * mval[hb])
                return carry

            lax.fori_loop(0, npb, pb_body, 0)

        def send(c, b):
            pltpu.async_copy(
                OUTB[b],
                w2_hbm.at[pl.ds(c * (BSTRIPE // 2), BSTRIPE // 2), :],
                SEM_O[b])

        def wait_send(c, b):
            pltpu.make_async_copy(
                OUTB[b],
                w2_hbm.at[pl.ds(c * (BSTRIPE // 2), BSTRIPE // 2), :],
                SEM_O[b]).wait()

        fire(wid, 0)
        fire(wid + NW, 1)

        nt = (NFULL - 1) // NW + 1

        def pair(h, carry):
            for b in (0, 1):
                t = 2 * h + b
                c = wid + t * NW

                @pl.when(c < NFULL)
                def _():
                    recv(c, b)

                    @pl.when(h >= 1)
                    def _():
                        wait_send(c - 2 * NW, b)

                    transp(b, BSTRIPE // 32)
                    send(c, b)

                    @pl.when(c + 2 * NW < NFULL)
                    def _():
                        fire(c + 2 * NW, b)
            return carry

        lax.fori_loop(0, (nt + 1) // 2, pair, 0)

        def waitlast(t, b):
            c = wid + t * NW

            @pl.when((c < NFULL) & (c + 2 * NW >= NFULL))
            def _():
                wait_send(c, b)

        for t in range(nt - 4, nt):
            if t >= 0:
                waitlast(t, t % 2)

        @pl.when(wid == NW - 1)
        def _():
            pltpu.sync_copy(tail_hbm, OUTB[0].at[pl.ds(0, TAILC // 2), :])
            pltpu.sync_copy(OUTB[0].at[pl.ds(0, TAILC // 2), :],
                            w2_hbm.at[pl.ds(NFULL * (BSTRIPE // 2),
                                            TAILC // 2), :])

    return ka


def _make_phase_b():
    mesh = plsc.VectorSubcoreMesh(
        core_axis_name="c", subcore_axis_name="s",
        num_cores=NC, num_subcores=NS,
    )

    @functools.partial(
        pl.kernel,
        mesh=mesh,
        compiler_params=_COMPILER_PARAMS,
        out_type=jax.ShapeDtypeStruct((L * DIM, B), jnp.float32),
        scratch_types=[
            pltpu.VMEM((BCHUNK,), jnp.int32),
            pltpu.VMEM((BCHUNK,), jnp.int32),
            pltpu.VMEM((BCHUNK,), jnp.int32),
            pltpu.VMEM((BCHUNK,), jnp.int32),
            pltpu.VMEM((BCHUNK, 2 * DIM), jnp.float32),
            pltpu.VMEM((BCHUNK, 2 * DIM), jnp.float32),
            pltpu.VMEM((DIM, BCHUNK), jnp.float32),
            pltpu.VMEM((DIM, BCHUNK), jnp.float32),
            pltpu.SemaphoreType.DMA,
            pltpu.SemaphoreType.DMA,
            pltpu.SemaphoreType.DMA,
            pltpu.SemaphoreType.DMA,
        ],
    )
    def kb(words_hbm, table_hbm, out_hbm, idx0, idx1, pidx0, pidx1,
           rows0, rows1, trans0, trans1, sem_r0, sem_r1, sem_w0, sem_w1):
        IDX = (idx0, idx1)
        PIDX = (pidx0, pidx1)
        ROWS = (rows0, rows1)
        TRANS = (trans0, trans1)
        SEM_R = (sem_r0, sem_r1)
        SEM_W = (sem_w0, sem_w1)
        wid = lax.axis_index("s") * NC + lax.axis_index("c")
        b0 = wid * BPW
        lane = lax.iota(jnp.int32, LANES)

        def fire(c, b):
            l = lax.shift_right_logical(c, 1)
            boff = b0 + (c & 1) * BCHUNK
            src = l * B + boff
            ib = IDX[b]
            pb = PIDX[b]
            pltpu.sync_copy(words_hbm.at[pl.ds(src, BCHUNK)], ib)
            for i in range(BCHUNK // LANES):
                sl = pl.ds(i * LANES, LANES)
                pb[sl] = lax.shift_right_logical(ib[sl], 1)
            pltpu.async_copy(table_hbm.at[pb], ROWS[b], SEM_R[b])

        def recv(c, b):
            pltpu.make_async_copy(table_hbm.at[PIDX[b]], ROWS[b],
                                  SEM_R[b]).wait()

        def transpose(c, b):
            rows = ROWS[b]
            trans = TRANS[b]
            ix = IDX[b]

            def colgrp(g8, carry):
                rb = []
                for u in range(4):
                    rbase = pl.multiple_of((4 * g8 + u) * LANES, LANES)
                    rows16 = rbase + lane
                    col0 = (ix[pl.ds(rbase, LANES)] & 1) * DIM
                    rb.append((rows16, col0))
                for kk in range(LANES):
                    diag = (lane + kk) & (LANES - 1)
                    for db in range(DIM // LANES):
                        dvec = db * LANES + diag
                        vals = []
                        for rows16, col0 in rb:
                            vals.append(plsc.load_gather(
                                rows, [rows16, col0 + dvec]))
                        for (rows16, col0), v in zip(rb, vals):
                            plsc.store_scatter(trans, [dvec, rows16], v)
                return carry

            lax.fori_loop(0, BCHUNK // (4 * LANES), colgrp, 0)

        def send(c, b):
            l = lax.shift_right_logical(c, 1)
            boff = b0 + (c & 1) * BCHUNK
            pltpu.async_copy(
                TRANS[b],
                out_hbm.at[pl.ds(l * DIM, DIM), pl.ds(boff, BCHUNK)],
                SEM_W[b])

        def wait_send(c, b):
            l = lax.shift_right_logical(c, 1)
            boff = b0 + (c & 1) * BCHUNK
            pltpu.make_async_copy(
                TRANS[b],
                out_hbm.at[pl.ds(l * DIM, DIM), pl.ds(boff, BCHUNK)],
                SEM_W[b]).wait()

        fire(0, 0)
        fire(1, 1)

        def pair(h, carry):
            for b in (0, 1):
                c = 2 * h + b
                recv(c, b)

                @pl.when(h >= 1)
                def _():
                    wait_send(c - 2, b)

                transpose(c, b)
                send(c, b)

                @pl.when(c + 2 < NCHUNK)
                def _():
                    fire(c + 2, b)
            return carry

        lax.fori_loop(0, NCHUNK // 2, pair, 0)
        wait_send(NCHUNK - 2, 0)
        wait_send(NCHUNK - 1, 1)

    return kb


_PHASE_A = _make_phase_a()
_PHASE_B = _make_phase_b()


@jax.jit
def kernel(words, raw_weight, mask):
    flat_words = words.T.reshape(-1).astype(jnp.int32)
    flat_mask = mask.reshape(-1)
    tail = (raw_weight[NFULL * BSTRIPE:, :]
            * mask[NFULL * BSTRIPE:]).reshape(TAILC // 2, 2 * DIM)
    w2 = _PHASE_A(raw_weight.T, flat_mask, tail)
    out2 = _PHASE_B(flat_words, w2)
    return out2.reshape(L, DIM, B).transpose(2, 0, 1)


# speedup vs baseline: 1.0655x; 1.0655x over previous
import functools

import jax
import jax.numpy as jnp
from jax import lax
from jax.experimental import pallas as pl
from jax.experimental.pallas import tpu as pltpu
from jax.experimental.pallas import tpu_sc as plsc

VOCAB = 1000000
DIM = 64
B = 16384
L = 50
NC = 2
NS = 16
NW = NC * NS
LANES = 16
BPW = B // NW
BCHUNK = 256
NCHUNK = L * BPW // BCHUNK
BSTRIPE = 256
NFULL = VOCAB // BSTRIPE
TAILC = VOCAB - NFULL * BSTRIPE

_COMPILER_PARAMS = pltpu.CompilerParams(
    use_tc_tiling_on_sc=True, needs_layout_passes=False,
    disable_bounds_checks=True)


def _make_phase_a():
    mesh = plsc.VectorSubcoreMesh(
        core_axis_name="c", subcore_axis_name="s",
        num_cores=NC, num_subcores=NS,
    )

    @functools.partial(
        pl.kernel,
        mesh=mesh,
        compiler_params=_COMPILER_PARAMS,
        out_type=jax.ShapeDtypeStruct((VOCAB // 2, 2 * DIM), jnp.float32),
        scratch_types=[
            pltpu.VMEM((DIM, BSTRIPE), jnp.float32),
            pltpu.VMEM((DIM, BSTRIPE), jnp.float32),
            pltpu.VMEM((BSTRIPE,), jnp.float32),
            pltpu.VMEM((BSTRIPE,), jnp.float32),
            pltpu.VMEM((BSTRIPE // 2, 2 * DIM), jnp.float32),
            pltpu.VMEM((BSTRIPE // 2, 2 * DIM), jnp.float32),
            pltpu.SemaphoreType.DMA,
            pltpu.SemaphoreType.DMA,
            pltpu.SemaphoreType.DMA,
            pltpu.SemaphoreType.DMA,
            pltpu.SemaphoreType.DMA,
            pltpu.SemaphoreType.DMA,
        ],
    )
    def ka(tt_hbm, maskf_hbm, tail_hbm, w2_hbm, in0, in1, mv0, mv1,
           ob0, ob1, sem_i0, sem_i1, sem_v0, sem_v1, sem_o0, sem_o1):
        INB = (in0, in1)
        MVB = (mv0, mv1)
        OUTB = (ob0, ob1)
        SEM_I = (sem_i0, sem_i1)
        SEM_V = (sem_v0, sem_v1)
        SEM_O = (sem_o0, sem_o1)
        wid = lax.axis_index("s") * NC + lax.axis_index("c")
        lane = lax.iota(jnp.int32, LANES)

        def fire(c, b):
            pltpu.async_copy(tt_hbm.at[:, pl.ds(c * BSTRIPE, BSTRIPE)],
                             INB[b], SEM_I[b])
            pltpu.async_copy(maskf_hbm.at[pl.ds(c * BSTRIPE, BSTRIPE)],
                             MVB[b], SEM_V[b])

        def recv(c, b):
            pltpu.make_async_copy(tt_hbm.at[:, pl.ds(c * BSTRIPE, BSTRIPE)],
                                  INB[b], SEM_I[b]).wait()
            pltpu.make_async_copy(maskf_hbm.at[pl.ds(c * BSTRIPE, BSTRIPE)],
                                  MVB[b], SEM_V[b]).wait()

        def transp(b, npb):
            inb, mvb, outb = INB[b], MVB[b], OUTB[b]

            def pb_body(pb, carry):
                prow = pb * LANES + lane
                cvec = 32 * pb + 2 * lane
                mval = (plsc.load_gather(mvb, [cvec]),
                        plsc.load_gather(mvb, [cvec + 1]))
                for kk in range(LANES):
                    diag = (lane + kk) & (LANES - 1)
                    vs = []
                    for hb in (0, 1):
                        for cq in range(4):
                            vs.append((hb, cq, plsc.load_gather(
                                inb, [cq * LANES + diag, cvec + hb])))
                    for hb, cq, v in vs:
                        plsc.store_scatter(
                            outb, [prow, (4 * hb + cq) * LANES + diag],
                            v * mval[hb])
                return carry

            lax.fori_loop(0, npb, pb_body, 0)

        def send(c, b):
            pltpu.async_copy(
                OUTB[b],
                w2_hbm.at[pl.ds(c * (BSTRIPE // 2), BSTRIPE // 2), :],
                SEM_O[b])

        def wait_send(c, b):
            pltpu.make_async_copy(
                OUTB[b],
                w2_hbm.at[pl.ds(c * (BSTRIPE // 2), BSTRIPE // 2), :],
                SEM_O[b]).wait()

        fire(wid, 0)
        fire(wid + NW, 1)

        nt = (NFULL - 1) // NW + 1

        def pair(h, carry):
            for b in (0, 1):
                t = 2 * h + b
                c = wid + t * NW

                @pl.when(c < NFULL)
                def _():
                    recv(c, b)

                    @pl.when(h >= 1)
                    def _():
                        wait_send(c - 2 * NW, b)

                    transp(b, BSTRIPE // 32)
                    send(c, b)

                    @pl.when(c + 2 * NW < NFULL)
                    def _():
                        fire(c + 2 * NW, b)
            return carry

        lax.fori_loop(0, (nt + 1) // 2, pair, 0)

        def waitlast(t, b):
            c = wid + t * NW

            @pl.when((c < NFULL) & (c + 2 * NW >= NFULL))
            def _():
                wait_send(c, b)

        for t in range(nt - 4, nt):
            if t >= 0:
                waitlast(t, t % 2)

        @pl.when(wid == NW - 1)
        def _():
            pltpu.sync_copy(tail_hbm, OUTB[0].at[pl.ds(0, TAILC // 2), :])
            pltpu.sync_copy(OUTB[0].at[pl.ds(0, TAILC // 2), :],
                            w2_hbm.at[pl.ds(NFULL * (BSTRIPE // 2),
                                            TAILC // 2), :])

    return ka


def _make_phase_b():
    mesh = plsc.VectorSubcoreMesh(
        core_axis_name="c", subcore_axis_name="s",
        num_cores=NC, num_subcores=NS,
    )

    @functools.partial(
        pl.kernel,
        mesh=mesh,
        compiler_params=_COMPILER_PARAMS,
        out_type=jax.ShapeDtypeStruct((L * DIM, B), jnp.float32),
        scratch_types=[
            pltpu.VMEM((BCHUNK,), jnp.int32),
            pltpu.VMEM((BCHUNK,), jnp.int32),
            pltpu.VMEM((BCHUNK,), jnp.int32),
            pltpu.VMEM((BCHUNK,), jnp.int32),
            pltpu.VMEM((BCHUNK, 2 * DIM), jnp.float32),
            pltpu.VMEM((BCHUNK, 2 * DIM), jnp.float32),
            pltpu.VMEM((DIM, BCHUNK), jnp.float32),
            pltpu.VMEM((DIM, BCHUNK), jnp.float32),
            pltpu.SemaphoreType.DMA,
            pltpu.SemaphoreType.DMA,
            pltpu.SemaphoreType.DMA,
            pltpu.SemaphoreType.DMA,
        ],
    )
    def kb(words_hbm, table_hbm, out_hbm, idx0, idx1, pidx0, pidx1,
           rows0, rows1, trans0, trans1, sem_r0, sem_r1, sem_w0, sem_w1):
        IDX = (idx0, idx1)
        PIDX = (pidx0, pidx1)
        ROWS = (rows0, rows1)
        TRANS = (trans0, trans1)
        SEM_R = (sem_r0, sem_r1)
        SEM_W = (sem_w0, sem_w1)
        wid = lax.axis_index("s") * NC + lax.axis_index("c")
        b0 = wid * BPW
        lane = lax.iota(jnp.int32, LANES)

        def fire(c, b):
            l = lax.shift_right_logical(c, 1)
            boff = b0 + (c & 1) * BCHUNK
            src = l * B + boff
            ib = IDX[b]
            pb = PIDX[b]
            pltpu.sync_copy(words_hbm.at[pl.ds(src, BCHUNK)], ib)
            for i in range(BCHUNK // LANES):
                sl = pl.ds(i * LANES, LANES)
                pb[sl] = lax.shift_right_logical(ib[sl], 1)
            pltpu.async_copy(table_hbm.at[pb], ROWS[b], SEM_R[b])

        def recv(c, b):
            pltpu.make_async_copy(table_hbm.at[PIDX[b]], ROWS[b],
                                  SEM_R[b]).wait()

        def transpose(c, b):
            rows = ROWS[b]
            trans = TRANS[b]
            ix = IDX[b]

            def colgrp(g8, carry):
                rb = []
                for u in range(4):
                    rbase = pl.multiple_of((4 * g8 + u) * LANES, LANES)
                    rows16 = rbase + lane
                    col0 = (ix[pl.ds(rbase, LANES)] & 1) * DIM
                    rb.append((rows16, col0))
                for kk in range(LANES):
                    diag = (lane + kk) & (LANES - 1)
                    for db in range(DIM // LANES):
                        dvec = db * LANES + diag
                        vals = []
                        for rows16, col0 in rb:
                            vals.append(plsc.load_gather(
                                rows, [rows16, col0 + dvec]))
                        for (rows16, col0), v in zip(rb, vals):
                            plsc.store_scatter(trans, [dvec, rows16], v)
                return carry

            lax.fori_loop(0, BCHUNK // (4 * LANES), colgrp, 0)

        def send(c, b):
            l = lax.shift_right_logical(c, 1)
            boff = b0 + (c & 1) * BCHUNK
            pltpu.async_copy(
                TRANS[b],
                out_hbm.at[pl.ds(l * DIM, DIM), pl.ds(boff, BCHUNK)],
                SEM_W[b])

        def wait_send(c, b):
            l = lax.shift_right_logical(c, 1)
            boff = b0 + (c & 1) * BCHUNK
            pltpu.make_async_copy(
                TRANS[b],
                out_hbm.at[pl.ds(l * DIM, DIM), pl.ds(boff, BCHUNK)],
                SEM_W[b]).wait()

        fire(0, 0)
        fire(1, 1)

        def pair(h, carry):
            for b in (0, 1):
                c = 2 * h + b
                recv(c, b)

                @pl.when(h >= 1)
                def _():
                    wait_send(c - 2, b)

                transpose(c, b)
                send(c, b)

                @pl.when(c + 2 < NCHUNK)
                def _():
                    fire(c + 2, b)
            return carry

        lax.fori_loop(0, NCHUNK // 2, pair, 0)
        wait_send(NCHUNK - 2, 0)
        wait_send(NCHUNK - 1, 1)

    return kb


_PHASE_A = _make_phase_a()
_PHASE_B = _make_phase_b()


@jax.jit
def kernel(words, raw_weight, mask):
    flat_words = words.T.reshape(-1).astype(jnp.int32)
    flat_mask = mask.reshape(-1)
    tail = (raw_weight[NFULL * BSTRIPE:, :]
            * mask[NFULL * BSTRIPE:]).reshape(TAILC // 2, 2 * DIM)
    w2 = _PHASE_A(raw_weight.T, flat_mask, tail)
    out2 = _PHASE_B(flat_words, w2)
    return out2.reshape(L, DIM, B).transpose(2, 0, 1)
